# Initial kernel scaffold; baseline (speedup 1.0000x reference)
import jax
import jax.numpy as jnp
from jax.experimental import pallas as pl


def kernel(x, adj, srcs, drts, W):
    raise NotImplementedError("write your pallas kernel here")


# speedup vs baseline: 1.2911x; 1.2911x over previous
import functools

import jax
import jax.numpy as jnp
from jax import lax
from jax.experimental import pallas as pl
from jax.experimental.pallas import tpu as pltpu
from jax.experimental.pallas import tpu_sc as plsc

N_NODES = 10000
D = 128
E = 320000

NC = 2
NS = 16
L = 16
NW = NC * NS

CHUNK = 128
NCHUNKS = -(-E // (NW * CHUNK))
EPW = NCHUNKS * CHUNK
E_PAD = EPW * NW

N_PAD = 10112
ROWS_PER_TILE = N_PAD // NS
GARBAGE_ROW = N_NODES

_MESH = plsc.VectorSubcoreMesh(
    core_axis_name="c", subcore_axis_name="s", num_cores=NC, num_subcores=NS)


def _k1_body(x_hbm, src_hbm, dst_hbm, z128_hbm, z16_hbm, agg_out, deg_out,
             sidx_v, didx_v, rows_v, ones_v, sem, agg_sh, deg_sh):
    c = lax.axis_index("c")
    s = lax.axis_index("s")
    wid = c * NS + s
    r0 = s * ROWS_PER_TILE
    pltpu.sync_copy(z128_hbm.at[pl.ds(r0, ROWS_PER_TILE)],
                    agg_sh.at[pl.ds(r0, ROWS_PER_TILE)])
    pltpu.sync_copy(z16_hbm.at[pl.ds(r0, ROWS_PER_TILE)],
                    deg_sh.at[pl.ds(r0, ROWS_PER_TILE)])
    one = lax.iota(jnp.int32, L) * 0.0 + 1.0
    one = one.astype(jnp.float32)

    def _ones_body(i, _):
        ones_v[i] = one
        return 0

    lax.fori_loop(0, CHUNK, _ones_body, 0)
    plsc.subcore_barrier()

    ebase = wid * EPW

    def _chunk_body(j, _):
        off = pl.multiple_of(ebase + j * CHUNK, CHUNK)
        pltpu.sync_copy(src_hbm.at[pl.ds(off, CHUNK)], sidx_v)
        pltpu.sync_copy(dst_hbm.at[pl.ds(off, CHUNK)], didx_v)
        pltpu.async_copy(x_hbm.at[sidx_v], rows_v, sem).wait()
        pltpu.sync_copy(rows_v, agg_sh.at[didx_v], add=True)
        pltpu.sync_copy(ones_v, deg_sh.at[didx_v], add=True)
        return 0

    lax.fori_loop(0, NCHUNKS, _chunk_body, 0)
    plsc.subcore_barrier()
    pltpu.sync_copy(agg_sh.at[pl.ds(r0, ROWS_PER_TILE)],
                    agg_out.at[c, pl.ds(r0, ROWS_PER_TILE)])
    pltpu.sync_copy(deg_sh.at[pl.ds(r0, ROWS_PER_TILE)],
                    deg_out.at[c, pl.ds(r0, ROWS_PER_TILE)])


_k1 = pl.kernel(
    _k1_body,
    out_type=(jax.ShapeDtypeStruct((NC, N_PAD, D), jnp.float32),
              jax.ShapeDtypeStruct((NC, N_PAD, L), jnp.float32)),
    mesh=_MESH,
    scratch_types=[
        pltpu.VMEM((CHUNK,), jnp.int32),
        pltpu.VMEM((CHUNK,), jnp.int32),
        pltpu.VMEM((CHUNK, D), jnp.float32),
        pltpu.VMEM((CHUNK, L), jnp.float32),
        pltpu.SemaphoreType.DMA,
        pltpu.VMEM_SHARED((N_PAD, D), jnp.float32),
        pltpu.VMEM_SHARED((N_PAD, L), jnp.float32),
    ],
)


def _k2_body(x_ref, a0_ref, a1_ref, d0_ref, d1_ref, w_ref, hn_ref):
    deg = d0_ref[:, 0:1] + d1_ref[:, 0:1]
    t = x_ref[...] + (a0_ref[...] + a1_ref[...]) / jnp.maximum(deg, 1.0)
    h = jnp.dot(t, w_ref[...], preferred_element_type=jnp.float32)
    nrm = jnp.sqrt(jnp.sum(h * h, axis=1, keepdims=True))
    hn_ref[...] = h / jnp.maximum(nrm, 1e-8)


_K2_BLK = 1000


def _k2(x, a0, a1, d0, d1, W):
    grid = (N_NODES // _K2_BLK,)
    return pl.pallas_call(
        _k2_body,
        grid=grid,
        in_specs=[
            pl.BlockSpec((_K2_BLK, D), lambda i: (i, 0)),
            pl.BlockSpec((_K2_BLK, D), lambda i: (i, 0)),
            pl.BlockSpec((_K2_BLK, D), lambda i: (i, 0)),
            pl.BlockSpec((_K2_BLK, L), lambda i: (i, 0)),
            pl.BlockSpec((_K2_BLK, L), lambda i: (i, 0)),
            pl.BlockSpec((D, D), lambda i: (0, 0)),
        ],
        out_specs=pl.BlockSpec((_K2_BLK, D), lambda i: (i, 0)),
        out_shape=jax.ShapeDtypeStruct((N_NODES, D), jnp.float32),
    )(x, a0, a1, d0, d1, W)


def _lane_permute(v, perm):
    return lax.gather(
        v, perm[:, None],
        lax.GatherDimensionNumbers(
            offset_dims=(), collapsed_slice_dims=(0,), start_index_map=(0,)),
        (1,), mode=lax.GatherScatterMode.PROMISE_IN_BOUNDS)


GROUPS = CHUNK // L


def _k3_body(hn_hbm, src_hbm, dst_hbm, out_hbm,
             sidx_v, didx_v, a_v, b_v, o_v, sem_a, sem_b):
    c = lax.axis_index("c")
    s = lax.axis_index("s")
    wid = c * NS + s
    ebase = wid * EPW
    lanes = lax.iota(jnp.int32, L)
    perms = [lax.rem(lanes + sh, L) for sh in (8, 4, 2, 1)]

    def _chunk_body(j, _):
        off = pl.multiple_of(ebase + j * CHUNK, CHUNK)
        pltpu.sync_copy(src_hbm.at[pl.ds(off, CHUNK)], sidx_v)
        pltpu.sync_copy(dst_hbm.at[pl.ds(off, CHUNK)], didx_v)
        cp_a = pltpu.async_copy(hn_hbm.at[sidx_v], a_v, sem_a)
        cp_b = pltpu.async_copy(hn_hbm.at[didx_v], b_v, sem_b)
        cp_a.wait()
        cp_b.wait()

        def _group_body(g, _):
            outv = lanes * 0.0
            for el in range(L):
                e = g * L + el
                acc = a_v[e, pl.ds(0, L)] * b_v[e, pl.ds(0, L)]
                for q in range(1, D // L):
                    acc = acc + (a_v[e, pl.ds(q * L, L)]
                                 * b_v[e, pl.ds(q * L, L)])
                for perm in perms:
                    acc = acc + _lane_permute(acc, perm)
                outv = jnp.where(lanes == el, acc, outv)
            o_v[g] = outv * 0.5 + 0.5
            return 0

        lax.fori_loop(0, GROUPS, _group_body, 0)
        row_off = pl.multiple_of((ebase + j * CHUNK) // L, GROUPS)
        pltpu.sync_copy(o_v, out_hbm.at[pl.ds(row_off, GROUPS)])
        return 0

    lax.fori_loop(0, NCHUNKS, _chunk_body, 0)


_k3 = pl.kernel(
    _k3_body,
    out_type=jax.ShapeDtypeStruct((E_PAD // L, L), jnp.float32),
    mesh=_MESH,
    scratch_types=[
        pltpu.VMEM((CHUNK,), jnp.int32),
        pltpu.VMEM((CHUNK,), jnp.int32),
        pltpu.VMEM((CHUNK, D), jnp.float32),
        pltpu.VMEM((CHUNK, D), jnp.float32),
        pltpu.VMEM((GROUPS, L), jnp.float32),
        pltpu.SemaphoreType.DMA,
        pltpu.SemaphoreType.DMA,
    ],
)


def kernel(x, adj, srcs, drts, W):
    src = adj[0].astype(jnp.int32)
    dst = adj[1].astype(jnp.int32)
    pad = E_PAD - E
    sp = jnp.concatenate([srcs.astype(jnp.int32), jnp.zeros((pad,), jnp.int32)])
    dp = jnp.concatenate([drts.astype(jnp.int32), jnp.zeros((pad,), jnp.int32)])
    agg = jax.ops.segment_sum(x[src], dst, num_segments=N_NODES)
    deg = jax.ops.segment_sum(jnp.ones((E,), jnp.float32), dst,
                              num_segments=N_NODES)
    zD = jnp.zeros((N_NODES, D), jnp.float32)
    zL = jnp.zeros((N_NODES, L), jnp.float32)
    degL = jnp.broadcast_to(deg[:, None], (N_NODES, L))
    hn = _k2(x, agg, zD, degL, zL, W)
    out = _k3(hn, sp, dp)
    return out.reshape(-1)[:E]


# speedup vs baseline: 2.7722x; 2.1472x over previous
import functools

import jax
import jax.numpy as jnp
from jax import lax
from jax.experimental import pallas as pl
from jax.experimental.pallas import tpu as pltpu
from jax.experimental.pallas import tpu_sc as plsc

N_NODES = 10000
D = 128
E = 320000

NC = 2
NS = 16
L = 16
NW = NC * NS

CHUNK = 128
NCHUNKS = -(-E // (NW * CHUNK))
EPW = NCHUNKS * CHUNK
E_PAD = EPW * NW

N_PAD = 10112
ROWS_PER_TILE = N_PAD // NS
GARBAGE_ROW = N_NODES

_MESH = plsc.VectorSubcoreMesh(
    core_axis_name="c", subcore_axis_name="s", num_cores=NC, num_subcores=NS)


STRIDE = 640
WCHUNKS = STRIDE // CHUNK


def _k1_body(x_hbm, src_hbm, dst_hbm, zrows_hbm, zidx_hbm, agg_out,
             sidx_v, didx_v, idx_v, rows_v, sem, agg_sh):
    c = lax.axis_index("c")
    s = lax.axis_index("s")
    wid = c * NS + s

    pltpu.sync_copy(zrows_hbm, rows_v)

    def _zero_blk(k, _):
        zoff = pl.multiple_of(s * STRIDE + k * CHUNK, CHUNK)
        pltpu.sync_copy(zidx_hbm.at[pl.ds(zoff, CHUNK)], idx_v)
        pltpu.sync_copy(rows_v, agg_sh.at[idx_v])
        return 0

    lax.fori_loop(0, WCHUNKS, _zero_blk, 0)
    plsc.subcore_barrier()

    ebase = wid * EPW

    def _chunk_body(j, _):
        off = pl.multiple_of(ebase + j * CHUNK, CHUNK)
        pltpu.sync_copy(src_hbm.at[pl.ds(off, CHUNK)], sidx_v)
        pltpu.sync_copy(dst_hbm.at[pl.ds(off, CHUNK)], didx_v)
        pltpu.async_copy(x_hbm.at[sidx_v], rows_v, sem).wait()
        pltpu.sync_copy(rows_v, agg_sh.at[didx_v], add=True)
        return 0

    lax.fori_loop(0, NCHUNKS, _chunk_body, 0)
    plsc.subcore_barrier()

    def _wb_blk(k, _):
        zoff = pl.multiple_of(s * STRIDE + k * CHUNK, CHUNK)
        pltpu.sync_copy(zidx_hbm.at[pl.ds(zoff, CHUNK)], idx_v)
        pltpu.async_copy(agg_sh.at[idx_v], rows_v, sem).wait()
        pltpu.sync_copy(rows_v, agg_out.at[c, pl.ds(zoff, CHUNK)])
        return 0

    lax.fori_loop(0, WCHUNKS, _wb_blk, 0)


_k1 = pl.kernel(
    _k1_body,
    out_type=jax.ShapeDtypeStruct((NC, NS * STRIDE, D), jnp.float32),
    mesh=_MESH,
    scratch_types=[
        pltpu.VMEM((CHUNK,), jnp.int32),
        pltpu.VMEM((CHUNK,), jnp.int32),
        pltpu.VMEM((CHUNK,), jnp.int32),
        pltpu.VMEM((CHUNK, D), jnp.float32),
        pltpu.SemaphoreType.DMA,
        pltpu.VMEM_SHARED((N_PAD, D), jnp.float32),
    ],
)


def _host_zidx():
    import numpy as _np
    rows = []
    for s in range(NS):
        base = s * ROWS_PER_TILE
        idx = list(range(base, base + ROWS_PER_TILE))
        idx += list(range(N_PAD - (STRIDE - ROWS_PER_TILE), N_PAD))
        rows.append(idx)
    return _np.asarray(rows, dtype=_np.int32).reshape(-1)


_ZIDX_NP = _host_zidx()


def _k2_body(x_ref, a0_ref, a1_ref, d0_ref, d1_ref, w_ref, hn_ref):
    deg = d0_ref[:, 0:1] + d1_ref[:, 0:1]
    t = x_ref[...] + (a0_ref[...] + a1_ref[...]) / jnp.maximum(deg, 1.0)
    h = jnp.dot(t, w_ref[...], preferred_element_type=jnp.float32)
    nrm = jnp.sqrt(jnp.sum(h * h, axis=1, keepdims=True))
    hn_ref[...] = h / jnp.maximum(nrm, 1e-8)


_K2_BLK = 1000


def _k2(x, a0, a1, d0, d1, W):
    grid = (N_NODES // _K2_BLK,)
    return pl.pallas_call(
        _k2_body,
        grid=grid,
        in_specs=[
            pl.BlockSpec((_K2_BLK, D), lambda i: (i, 0)),
            pl.BlockSpec((_K2_BLK, D), lambda i: (i, 0)),
            pl.BlockSpec((_K2_BLK, D), lambda i: (i, 0)),
            pl.BlockSpec((_K2_BLK, L), lambda i: (i, 0)),
            pl.BlockSpec((_K2_BLK, L), lambda i: (i, 0)),
            pl.BlockSpec((D, D), lambda i: (0, 0)),
        ],
        out_specs=pl.BlockSpec((_K2_BLK, D), lambda i: (i, 0)),
        out_shape=jax.ShapeDtypeStruct((N_NODES, D), jnp.float32),
    )(x, a0, a1, d0, d1, W)


def _lane_permute(v, perm):
    return lax.gather(
        v, perm[:, None],
        lax.GatherDimensionNumbers(
            offset_dims=(), collapsed_slice_dims=(0,), start_index_map=(0,)),
        (1,), mode=lax.GatherScatterMode.PROMISE_IN_BOUNDS)


GROUPS = CHUNK // L


def _k3_body(hn_hbm, src_hbm, dst_hbm, out_hbm,
             sidx_v, didx_v, a_v, b_v, o_v, sem_a, sem_b):
    c = lax.axis_index("c")
    s = lax.axis_index("s")
    wid = c * NS + s
    ebase = wid * EPW
    lanes = lax.iota(jnp.int32, L)
    perms = [lax.rem(lanes + sh, L) for sh in (8, 4, 2, 1)]

    def _chunk_body(j, _):
        off = pl.multiple_of(ebase + j * CHUNK, CHUNK)
        pltpu.sync_copy(src_hbm.at[pl.ds(off, CHUNK)], sidx_v)
        pltpu.sync_copy(dst_hbm.at[pl.ds(off, CHUNK)], didx_v)
        cp_a = pltpu.async_copy(hn_hbm.at[sidx_v], a_v, sem_a)
        cp_b = pltpu.async_copy(hn_hbm.at[didx_v], b_v, sem_b)
        cp_a.wait()
        cp_b.wait()

        def _group_body(g, _):
            outv = lanes * 0.0
            for el in range(L):
                e = g * L + el
                acc = a_v[e, pl.ds(0, L)] * b_v[e, pl.ds(0, L)]
                for q in range(1, D // L):
                    acc = acc + (a_v[e, pl.ds(q * L, L)]
                                 * b_v[e, pl.ds(q * L, L)])
                for perm in perms:
                    acc = acc + _lane_permute(acc, perm)
                outv = jnp.where(lanes == el, acc, outv)
            o_v[g] = outv * 0.5 + 0.5
            return 0

        lax.fori_loop(0, GROUPS, _group_body, 0)
        row_off = pl.multiple_of((ebase + j * CHUNK) // L, GROUPS)
        pltpu.sync_copy(o_v, out_hbm.at[pl.ds(row_off, GROUPS)])
        return 0

    lax.fori_loop(0, NCHUNKS, _chunk_body, 0)


_k3 = pl.kernel(
    _k3_body,
    out_type=jax.ShapeDtypeStruct((E_PAD // L, L), jnp.float32),
    mesh=_MESH,
    scratch_types=[
        pltpu.VMEM((CHUNK,), jnp.int32),
        pltpu.VMEM((CHUNK,), jnp.int32),
        pltpu.VMEM((CHUNK, D), jnp.float32),
        pltpu.VMEM((CHUNK, D), jnp.float32),
        pltpu.VMEM((GROUPS, L), jnp.float32),
        pltpu.SemaphoreType.DMA,
        pltpu.SemaphoreType.DMA,
    ],
)


def kernel(x, adj, srcs, drts, W):
    src = adj[0].astype(jnp.int32)
    dst = adj[1].astype(jnp.int32)
    pad = E_PAD - E
    src_p = jnp.concatenate([src, jnp.zeros((pad,), jnp.int32)])
    dst_p = jnp.concatenate([dst, jnp.full((pad,), GARBAGE_ROW, jnp.int32)])
    sp = jnp.concatenate([srcs.astype(jnp.int32), jnp.zeros((pad,), jnp.int32)])
    dp = jnp.concatenate([drts.astype(jnp.int32), jnp.zeros((pad,), jnp.int32)])

    zrows = jnp.zeros((CHUNK, D), jnp.float32)
    zidx = jnp.asarray(_ZIDX_NP)
    out1 = _k1(x, src_p, dst_p, zrows, zidx)
    v = out1.reshape(NC, NS, STRIDE, D)[:, :, :ROWS_PER_TILE]
    v = v.reshape(NC, N_PAD, D)[:, :N_NODES]

    deg = jax.ops.segment_sum(jnp.ones((E,), jnp.float32), dst,
                              num_segments=N_NODES)
    degL = jnp.broadcast_to(deg[:, None], (N_NODES, L))
    zL = jnp.zeros((N_NODES, L), jnp.float32)

    hn = _k2(x, v[0], v[1], degL, zL, W)
    out = _k3(hn, sp, dp)
    return out.reshape(-1)[:E]
